# baseline (speedup 1.0000x reference)
import jax
import jax.numpy as jnp
from jax import lax
from jax.experimental import pallas as pl
from jax.experimental.pallas import tpu as pltpu

B, C, H, W = 64, 3, 512, 512
PH, PW = 64, 64
NRB = H // PH


def _merge_body(rows_ref, cols_ref, img_ref, patch_ref, out_ref):
  b = pl.program_id(0)
  k = pl.program_id(1)
  r = rows_ref[b]
  c = cols_ref[b]
  base = k * PH

  out_ref[...] = img_ref[...]

  @pl.when((r < base + PH) & (r + PH > base))
  def _():
    patch = patch_ref[0]
    canvas = jnp.pad(patch, ((0, 0), (0, 0), (0, W - PW)))
    canvas = pltpu.roll(canvas, c, axis=2)
    canvas = pltpu.roll(canvas, r % PH, axis=1)

    row_ids = base + lax.broadcasted_iota(jnp.int32, (PH, W), 0)
    col_ids = lax.broadcasted_iota(jnp.int32, (PH, W), 1)
    inside = ((row_ids >= r) & (row_ids < r + PH)
              & (col_ids >= c) & (col_ids < c + PW))
    out_ref[0] = jnp.where(inside[None], canvas, img_ref[0])


def kernel(image, top_left_rows, top_left_cols, learned_patch):
  grid_spec = pltpu.PrefetchScalarGridSpec(
      num_scalar_prefetch=2,
      grid=(B, NRB),
      in_specs=[
          pl.BlockSpec((1, C, PH, W), lambda b, k, rows, cols: (b, 0, k, 0)),
          pl.BlockSpec((1, C, PH, PW), lambda b, k, rows, cols: (0, 0, 0, 0)),
      ],
      out_specs=pl.BlockSpec((1, C, PH, W),
                             lambda b, k, rows, cols: (b, 0, k, 0)),
  )
  return pl.pallas_call(
      _merge_body,
      grid_spec=grid_spec,
      out_shape=jax.ShapeDtypeStruct((B, C, H, W), jnp.float32),
  )(top_left_rows, top_left_cols, image, learned_patch)


# speedup vs baseline: 1.5915x; 1.5915x over previous
import jax
import jax.numpy as jnp
from jax import lax
from jax.experimental import pallas as pl
from jax.experimental.pallas import tpu as pltpu

B, C, H, W = 64, 3, 512, 512
PH, PW = 64, 64


def _merge_body(rows_ref, cols_ref, img_ref, patch_ref, out_ref):
  b = pl.program_id(0)
  c_idx = pl.program_id(1)
  r = rows_ref[b]
  c = cols_ref[b]

  patch = patch_ref[0, 0]
  canvas = jnp.pad(patch, ((0, H - PH), (0, W - PW)))
  canvas = pltpu.roll(canvas, c, axis=1)
  canvas = pltpu.roll(canvas, r, axis=0)

  row_ids = lax.broadcasted_iota(jnp.int32, (H, W), 0)
  col_ids = lax.broadcasted_iota(jnp.int32, (H, W), 1)
  inside = ((row_ids >= r) & (row_ids < r + PH)
            & (col_ids >= c) & (col_ids < c + PW))

  out_ref[0, 0] = jnp.where(inside, canvas, img_ref[0, 0])


def kernel(image, top_left_rows, top_left_cols, learned_patch):
  grid_spec = pltpu.PrefetchScalarGridSpec(
      num_scalar_prefetch=2,
      grid=(B, C),
      in_specs=[
          pl.BlockSpec((1, 1, H, W), lambda b, ch, rows, cols: (b, ch, 0, 0)),
          pl.BlockSpec((1, 1, PH, PW),
                       lambda b, ch, rows, cols: (0, ch, 0, 0)),
      ],
      out_specs=pl.BlockSpec((1, 1, H, W),
                             lambda b, ch, rows, cols: (b, ch, 0, 0)),
  )
  return pl.pallas_call(
      _merge_body,
      grid_spec=grid_spec,
      out_shape=jax.ShapeDtypeStruct((B, C, H, W), jnp.float32),
  )(top_left_rows, top_left_cols, image, learned_patch)


# speedup vs baseline: 2.4776x; 1.5568x over previous
import jax
import jax.numpy as jnp
from jax import lax
from jax.experimental import pallas as pl
from jax.experimental.pallas import tpu as pltpu

B, C, H, W = 64, 3, 512, 512
PH, PW = 64, 64


def _merge_body(rows_ref, cols_ref, img_ref, patch_ref, out_ref):
  b = pl.program_id(0)
  r = rows_ref[b]
  c = cols_ref[b]

  patch = patch_ref[0]
  canvas = jnp.pad(patch, ((0, 0), (0, H - PH), (0, W - PW)))
  canvas = pltpu.roll(canvas, c, axis=2)
  canvas = pltpu.roll(canvas, r, axis=1)

  row_ids = lax.broadcasted_iota(jnp.int32, (H, W), 0)
  col_ids = lax.broadcasted_iota(jnp.int32, (H, W), 1)
  inside = ((row_ids >= r) & (row_ids < r + PH)
            & (col_ids >= c) & (col_ids < c + PW))

  img = img_ref[0]
  out_ref[0] = jnp.where(inside[None], canvas, img)


def kernel(image, top_left_rows, top_left_cols, learned_patch):
  grid_spec = pltpu.PrefetchScalarGridSpec(
      num_scalar_prefetch=2,
      grid=(B,),
      in_specs=[
          pl.BlockSpec((1, C, H, W), lambda b, rows, cols: (b, 0, 0, 0)),
          pl.BlockSpec((1, C, PH, PW), lambda b, rows, cols: (0, 0, 0, 0)),
      ],
      out_specs=pl.BlockSpec((1, C, H, W), lambda b, rows, cols: (b, 0, 0, 0)),
  )
  return pl.pallas_call(
      _merge_body,
      grid_spec=grid_spec,
      out_shape=jax.ShapeDtypeStruct((B, C, H, W), jnp.float32),
  )(top_left_rows, top_left_cols, image, learned_patch)


# speedup vs baseline: 2.7842x; 1.1238x over previous
import jax
import jax.numpy as jnp
from jax import lax
from jax.experimental import pallas as pl
from jax.experimental.pallas import tpu as pltpu

B, C, H, W = 64, 3, 512, 512
PH, PW = 64, 64


_WR = PH + 8


def _merge_body(rows_ref, cols_ref, img_ref, patch_ref, out_ref):
  b = pl.program_id(0)
  r = rows_ref[b]
  c = cols_ref[b]

  out_ref[...] = img_ref[...]

  r0 = pl.multiple_of((r // 8) * 8, 8)
  dr = r - r0

  patch = patch_ref[0]
  canvas = jnp.pad(patch, ((0, 0), (0, _WR - PH), (0, W - PW)))
  canvas = pltpu.roll(canvas, c, axis=2)
  canvas = pltpu.roll(canvas, dr, axis=1)

  row_ids = lax.broadcasted_iota(jnp.int32, (_WR, W), 0)
  col_ids = lax.broadcasted_iota(jnp.int32, (_WR, W), 1)
  inside = ((row_ids >= dr) & (row_ids < dr + PH)
            & (col_ids >= c) & (col_ids < c + PW))

  window = img_ref[0, :, pl.ds(r0, _WR), :]
  out_ref[0, :, pl.ds(r0, _WR), :] = jnp.where(inside[None], canvas, window)


def kernel(image, top_left_rows, top_left_cols, learned_patch):
  grid_spec = pltpu.PrefetchScalarGridSpec(
      num_scalar_prefetch=2,
      grid=(B,),
      in_specs=[
          pl.BlockSpec((1, C, H, W), lambda b, rows, cols: (b, 0, 0, 0)),
          pl.BlockSpec((1, C, PH, PW), lambda b, rows, cols: (0, 0, 0, 0)),
      ],
      out_specs=pl.BlockSpec((1, C, H, W), lambda b, rows, cols: (b, 0, 0, 0)),
  )
  return pl.pallas_call(
      _merge_body,
      grid_spec=grid_spec,
      out_shape=jax.ShapeDtypeStruct((B, C, H, W), jnp.float32),
  )(top_left_rows, top_left_cols, image, learned_patch)


# speedup vs baseline: 2.8724x; 1.0317x over previous
import jax
import jax.numpy as jnp
from jax import lax
from jax.experimental import pallas as pl
from jax.experimental.pallas import tpu as pltpu

B, C, H, W = 64, 3, 512, 512
PH, PW = 64, 64
NB = 2
_WR = PH + 8


def _merge_body(rows_ref, cols_ref, img_ref, patch_ref, out_ref):
  g = pl.program_id(0)

  out_ref[...] = img_ref[...]

  for j in range(NB):
    b = g * NB + j
    r = rows_ref[b]
    c = cols_ref[b]

    r0 = pl.multiple_of((r // 8) * 8, 8)
    dr = r - r0

    patch = patch_ref[0]
    canvas = jnp.pad(patch, ((0, 0), (0, _WR - PH), (0, W - PW)))
    canvas = pltpu.roll(canvas, c, axis=2)
    canvas = pltpu.roll(canvas, dr, axis=1)

    row_ids = lax.broadcasted_iota(jnp.int32, (_WR, W), 0)
    col_ids = lax.broadcasted_iota(jnp.int32, (_WR, W), 1)
    inside = ((row_ids >= dr) & (row_ids < dr + PH)
              & (col_ids >= c) & (col_ids < c + PW))

    window = img_ref[j, :, pl.ds(r0, _WR), :]
    out_ref[j, :, pl.ds(r0, _WR), :] = jnp.where(inside[None], canvas, window)


def kernel(image, top_left_rows, top_left_cols, learned_patch):
  grid_spec = pltpu.PrefetchScalarGridSpec(
      num_scalar_prefetch=2,
      grid=(B // NB,),
      in_specs=[
          pl.BlockSpec((NB, C, H, W), lambda g, rows, cols: (g, 0, 0, 0)),
          pl.BlockSpec((1, C, PH, PW), lambda g, rows, cols: (0, 0, 0, 0)),
      ],
      out_specs=pl.BlockSpec((NB, C, H, W), lambda g, rows, cols: (g, 0, 0, 0)),
  )
  return pl.pallas_call(
      _merge_body,
      grid_spec=grid_spec,
      out_shape=jax.ShapeDtypeStruct((B, C, H, W), jnp.float32),
  )(top_left_rows, top_left_cols, image, learned_patch)


# speedup vs baseline: 2.9069x; 1.0120x over previous
import jax
import jax.numpy as jnp
from jax import lax
from jax.experimental import pallas as pl
from jax.experimental.pallas import tpu as pltpu

B, C, H, W = 64, 3, 512, 512
PH, PW = 64, 64
NB = 4
_WR = PH + 8


def _merge_body(rows_ref, cols_ref, img_ref, patch_ref, out_ref):
  g = pl.program_id(0)

  out_ref[...] = img_ref[...]

  for j in range(NB):
    b = g * NB + j
    r = rows_ref[b]
    c = cols_ref[b]

    r0 = pl.multiple_of((r // 8) * 8, 8)
    dr = r - r0

    patch = patch_ref[0]
    canvas = jnp.pad(patch, ((0, 0), (0, _WR - PH), (0, W - PW)))
    canvas = pltpu.roll(canvas, c, axis=2)
    canvas = pltpu.roll(canvas, dr, axis=1)

    row_ids = lax.broadcasted_iota(jnp.int32, (_WR, W), 0)
    col_ids = lax.broadcasted_iota(jnp.int32, (_WR, W), 1)
    inside = ((row_ids >= dr) & (row_ids < dr + PH)
              & (col_ids >= c) & (col_ids < c + PW))

    window = img_ref[j, :, pl.ds(r0, _WR), :]
    out_ref[j, :, pl.ds(r0, _WR), :] = jnp.where(inside[None], canvas, window)


def kernel(image, top_left_rows, top_left_cols, learned_patch):
  grid_spec = pltpu.PrefetchScalarGridSpec(
      num_scalar_prefetch=2,
      grid=(B // NB,),
      in_specs=[
          pl.BlockSpec((NB, C, H, W), lambda g, rows, cols: (g, 0, 0, 0)),
          pl.BlockSpec((1, C, PH, PW), lambda g, rows, cols: (0, 0, 0, 0)),
      ],
      out_specs=pl.BlockSpec((NB, C, H, W), lambda g, rows, cols: (g, 0, 0, 0)),
  )
  return pl.pallas_call(
      _merge_body,
      grid_spec=grid_spec,
      out_shape=jax.ShapeDtypeStruct((B, C, H, W), jnp.float32),
  )(top_left_rows, top_left_cols, image, learned_patch)
